# baseline (device time: 1502006 ns/iter reference)
import jax
import jax.numpy as jnp
from jax import lax
from jax.experimental import pallas as pl
from jax.experimental.pallas import tpu as pltpu

N_DEV = 16
NSUB = 4


def kernel(x, w_mat, scale_x, scale_w):
    m, _ = x.shape
    _, n = w_mat.shape
    ch = m // N_DEV
    chh = ch // NSUB
    nh = n // 2

    partial = jnp.dot(x, w_mat, preferred_element_type=jnp.int32)
    sc = (scale_x * scale_w).astype(jnp.float32).reshape(1, 1)

    def body(sc_ref, p_ref, out_ref,
             recv_f, recv_b, send_f, send_b, loc_f, loc_b, stg_f, stg_b,
             rss_f, rsr_f, rss_b, rsr_b,
             ags_f, agr_f, ags_b, agr_b,
             lsem_f, lsem_b, stsem_f, stsem_b, cred_f, cred_b):
        me = lax.axis_index("i")
        left = lax.rem(me + N_DEV - 1, N_DEV)
        right = lax.rem(me + 1, N_DEV)

        class Dir:
            pass

        fwd = Dir()
        fwd.dev, fwd.sign, fwd.cred_to, fwd.col = right, -1, left, pl.ds(0, nh)
        fwd.recv, fwd.send, fwd.loc, fwd.stg = recv_f, send_f, loc_f, stg_f
        fwd.rss, fwd.rsr, fwd.ags, fwd.agr = rss_f, rsr_f, ags_f, agr_f
        fwd.lsem, fwd.stsem, fwd.cred = lsem_f, stsem_f, cred_f
        bwd = Dir()
        bwd.dev, bwd.sign, bwd.cred_to, bwd.col = left, 1, right, pl.ds(nh, nh)
        bwd.recv, bwd.send, bwd.loc, bwd.stg = recv_b, send_b, loc_b, stg_b
        bwd.rss, bwd.rsr, bwd.ags, bwd.agr = rss_b, rsr_b, ags_b, agr_b
        bwd.lsem, bwd.stsem, bwd.cred = lsem_b, stsem_b, cred_b
        dirs = (fwd, bwd)

        def chunk(k, sign):
            return lax.rem(me + sign * k + 2 * N_DEV, N_DEV)

        def rows(c, sub):
            return pl.ds(c * ch + sub * chh, chh)

        def remote(src, dst, ssem, rsem, dev):
            return pltpu.make_async_remote_copy(
                src_ref=src, dst_ref=dst, send_sem=ssem, recv_sem=rsem,
                device_id=(dev,), device_id_type=pl.DeviceIdType.MESH)

        rcur = [[None] * NSUB for _ in dirs]
        cps = [[None] * NSUB for _ in dirs]
        for di, D in enumerate(dirs):
            for sub in range(NSUB):
                r = remote(p_ref.at[rows(chunk(0, D.sign), sub), D.col],
                           D.recv.at[0, sub],
                           D.rss.at[0, sub], D.rsr.at[0, sub], D.dev)
                r.start()
                rcur[di][sub] = r
                cp = pltpu.make_async_copy(
                    p_ref.at[rows(chunk(1, D.sign), sub), D.col],
                    D.loc.at[0, sub], D.lsem.at[0, sub])
                cp.start()
                cps[di][sub] = cp

        for s in range(N_DEV - 1):
            slot = s % 2
            for sub in range(NSUB):
                for di, D in enumerate(dirs):
                    rcur[di][sub].wait()
                    cps[di][sub].wait()
                    D.send[sub] = D.recv[slot, sub] + D.loc[slot, sub]
                    if s < N_DEV - 2:
                        if sub == 0 and s + 1 >= 2:
                            pl.semaphore_wait(D.cred, 1)
                        r = remote(D.send.at[sub],
                                   D.recv.at[(s + 1) % 2, sub],
                                   D.rss.at[s + 1, sub],
                                   D.rsr.at[s + 1, sub], D.dev)
                        r.start()
                        rcur[di][sub] = r
            for di, D in enumerate(dirs):
                if s <= N_DEV - 4:
                    pl.semaphore_signal(D.cred, inc=1,
                                        device_id=(D.cred_to,),
                                        device_id_type=pl.DeviceIdType.MESH)
                if s < N_DEV - 2:
                    for sub in range(NSUB):
                        cp = pltpu.make_async_copy(
                            p_ref.at[rows(chunk(s + 2, D.sign), sub), D.col],
                            D.loc.at[(s + 1) % 2, sub],
                            D.lsem.at[(s + 1) % 2, sub])
                        cp.start()
                        cps[di][sub] = cp

        rag = [[None] * NSUB for _ in dirs]
        sts = [[None] * NSUB for _ in dirs]
        for di, D in enumerate(dirs):
            rc = chunk(-1, D.sign)
            for sub in range(NSUB):
                D.stg[sub] = D.send[sub].astype(jnp.float32) * sc_ref[0, 0]
                r = remote(D.stg.at[sub], out_ref.at[rows(rc, sub), D.col],
                           D.ags.at[0, sub], D.agr.at[0, sub], D.dev)
                r.start()
                rag[di][sub] = r
                st = pltpu.make_async_copy(
                    D.stg.at[sub], out_ref.at[rows(rc, sub), D.col],
                    D.stsem.at[sub])
                st.start()
                sts[di][sub] = st

        for t in range(N_DEV - 1):
            for sub in range(NSUB):
                for di, D in enumerate(dirs):
                    rag[di][sub].wait()
                    if t < N_DEV - 2:
                        cr = rows(chunk(t, D.sign), sub)
                        r = remote(out_ref.at[cr, D.col],
                                   out_ref.at[cr, D.col],
                                   D.ags.at[t + 1, sub],
                                   D.agr.at[t + 1, sub], D.dev)
                        r.start()
                        rag[di][sub] = r
        for di, D in enumerate(dirs):
            for sub in range(NSUB):
                sts[di][sub].wait()

    nsem = N_DEV - 1
    return pl.pallas_call(
        body,
        out_shape=jax.ShapeDtypeStruct((m, n), jnp.float32),
        in_specs=[
            pl.BlockSpec(memory_space=pltpu.SMEM),
            pl.BlockSpec(memory_space=pl.ANY),
        ],
        out_specs=pl.BlockSpec(memory_space=pl.ANY),
        scratch_shapes=[
            pltpu.VMEM((2, NSUB, chh, nh), jnp.int32),
            pltpu.VMEM((2, NSUB, chh, nh), jnp.int32),
            pltpu.VMEM((NSUB, chh, nh), jnp.int32),
            pltpu.VMEM((NSUB, chh, nh), jnp.int32),
            pltpu.VMEM((2, NSUB, chh, nh), jnp.int32),
            pltpu.VMEM((2, NSUB, chh, nh), jnp.int32),
            pltpu.VMEM((NSUB, chh, nh), jnp.float32),
            pltpu.VMEM((NSUB, chh, nh), jnp.float32),
            pltpu.SemaphoreType.DMA((nsem, NSUB)),
            pltpu.SemaphoreType.DMA((nsem, NSUB)),
            pltpu.SemaphoreType.DMA((nsem, NSUB)),
            pltpu.SemaphoreType.DMA((nsem, NSUB)),
            pltpu.SemaphoreType.DMA((nsem, NSUB)),
            pltpu.SemaphoreType.DMA((nsem, NSUB)),
            pltpu.SemaphoreType.DMA((nsem, NSUB)),
            pltpu.SemaphoreType.DMA((nsem, NSUB)),
            pltpu.SemaphoreType.DMA((2, NSUB)),
            pltpu.SemaphoreType.DMA((2, NSUB)),
            pltpu.SemaphoreType.DMA((NSUB,)),
            pltpu.SemaphoreType.DMA((NSUB,)),
            pltpu.SemaphoreType.REGULAR,
            pltpu.SemaphoreType.REGULAR,
        ],
        compiler_params=pltpu.CompilerParams(
            vmem_limit_bytes=64 * 1024 * 1024,
        ),
    )(sc, partial)


# device time: 1456667 ns/iter; 1.0311x vs baseline; 1.0311x over previous
import jax
import jax.numpy as jnp
from jax import lax
from jax.experimental import pallas as pl
from jax.experimental.pallas import tpu as pltpu

N_DEV = 16
NSUB = 2


def kernel(x, w_mat, scale_x, scale_w):
    m, _ = x.shape
    _, n = w_mat.shape
    ch = m // N_DEV
    chh = ch // NSUB
    nh = n // 2

    sc = (scale_x * scale_w).astype(jnp.float32).reshape(1, 1)

    def body(sc_ref, x_ref, w_ref, out_ref,
             recv_f, recv_b, send_f, send_b, stg_f, stg_b,
             rss_f, rsr_f, rss_b, rsr_b,
             ags_f, agr_f, ags_b, agr_b,
             stsem_f, stsem_b, cred_f, cred_b):
        me = lax.axis_index("i")
        left = lax.rem(me + N_DEV - 1, N_DEV)
        right = lax.rem(me + 1, N_DEV)

        class Dir:
            pass

        fwd = Dir()
        fwd.dev, fwd.sign, fwd.cred_to, fwd.col = right, -1, left, pl.ds(0, nh)
        fwd.recv, fwd.send, fwd.stg = recv_f, send_f, stg_f
        fwd.rss, fwd.rsr, fwd.ags, fwd.agr = rss_f, rsr_f, ags_f, agr_f
        fwd.stsem, fwd.cred = stsem_f, cred_f
        bwd = Dir()
        bwd.dev, bwd.sign, bwd.cred_to, bwd.col = left, 1, right, pl.ds(nh, nh)
        bwd.recv, bwd.send, bwd.stg = recv_b, send_b, stg_b
        bwd.rss, bwd.rsr, bwd.ags, bwd.agr = rss_b, rsr_b, ags_b, agr_b
        bwd.stsem, bwd.cred = stsem_b, cred_b
        dirs = (fwd, bwd)

        def chunk(k, sign):
            return lax.rem(me + sign * k + 2 * N_DEV, N_DEV)

        def rows(c, sub):
            return pl.ds(c * ch + sub * chh, chh)

        def pgemm(c, sub, D):
            return jax.lax.dot_general(
                x_ref[rows(c, sub), :], w_ref[:, D.col],
                (((1,), (0,)), ((), ())),
                preferred_element_type=jnp.int32)

        def remote(src, dst, ssem, rsem, dev):
            return pltpu.make_async_remote_copy(
                src_ref=src, dst_ref=dst, send_sem=ssem, recv_sem=rsem,
                device_id=(dev,), device_id_type=pl.DeviceIdType.MESH)

        rcur = [[None] * NSUB for _ in dirs]
        for di, D in enumerate(dirs):
            for sub in range(NSUB):
                D.send[sub] = pgemm(chunk(0, D.sign), sub, D)
                r = remote(D.send.at[sub], D.recv.at[0, sub],
                           D.rss.at[0, sub], D.rsr.at[0, sub], D.dev)
                r.start()
                rcur[di][sub] = r

        for s in range(N_DEV - 1):
            slot = s % 2
            for sub in range(NSUB):
                for di, D in enumerate(dirs):
                    rcur[di][sub].wait()
                    D.send[sub] = (D.recv[slot, sub]
                                   + pgemm(chunk(s + 1, D.sign), sub, D))
                    if s < N_DEV - 2:
                        if sub == 0 and s + 1 >= 2:
                            pl.semaphore_wait(D.cred, 1)
                        r = remote(D.send.at[sub],
                                   D.recv.at[(s + 1) % 2, sub],
                                   D.rss.at[s + 1, sub],
                                   D.rsr.at[s + 1, sub], D.dev)
                        r.start()
                        rcur[di][sub] = r
            for D in dirs:
                if s <= N_DEV - 4:
                    pl.semaphore_signal(D.cred, inc=1,
                                        device_id=(D.cred_to,),
                                        device_id_type=pl.DeviceIdType.MESH)

        rag = [[None] * NSUB for _ in dirs]
        sts = [[None] * NSUB for _ in dirs]
        for di, D in enumerate(dirs):
            rc = chunk(-1, D.sign)
            for sub in range(NSUB):
                D.stg[sub] = D.send[sub].astype(jnp.float32) * sc_ref[0, 0]
                r = remote(D.stg.at[sub], out_ref.at[rows(rc, sub), D.col],
                           D.ags.at[0, sub], D.agr.at[0, sub], D.dev)
                r.start()
                rag[di][sub] = r
                st = pltpu.make_async_copy(
                    D.stg.at[sub], out_ref.at[rows(rc, sub), D.col],
                    D.stsem.at[sub])
                st.start()
                sts[di][sub] = st

        for t in range(N_DEV - 1):
            for sub in range(NSUB):
                for di, D in enumerate(dirs):
                    rag[di][sub].wait()
                    if t < N_DEV - 2:
                        cr = rows(chunk(t, D.sign), sub)
                        r = remote(out_ref.at[cr, D.col],
                                   out_ref.at[cr, D.col],
                                   D.ags.at[t + 1, sub],
                                   D.agr.at[t + 1, sub], D.dev)
                        r.start()
                        rag[di][sub] = r
        for di, D in enumerate(dirs):
            for sub in range(NSUB):
                sts[di][sub].wait()

    nsem = N_DEV - 1
    return pl.pallas_call(
        body,
        out_shape=jax.ShapeDtypeStruct((m, n), jnp.float32),
        in_specs=[
            pl.BlockSpec(memory_space=pltpu.SMEM),
            pl.BlockSpec(memory_space=pltpu.VMEM),
            pl.BlockSpec(memory_space=pltpu.VMEM),
        ],
        out_specs=pl.BlockSpec(memory_space=pl.ANY),
        scratch_shapes=[
            pltpu.VMEM((2, NSUB, chh, nh), jnp.int32),
            pltpu.VMEM((2, NSUB, chh, nh), jnp.int32),
            pltpu.VMEM((NSUB, chh, nh), jnp.int32),
            pltpu.VMEM((NSUB, chh, nh), jnp.int32),
            pltpu.VMEM((NSUB, chh, nh), jnp.float32),
            pltpu.VMEM((NSUB, chh, nh), jnp.float32),
            pltpu.SemaphoreType.DMA((nsem, NSUB)),
            pltpu.SemaphoreType.DMA((nsem, NSUB)),
            pltpu.SemaphoreType.DMA((nsem, NSUB)),
            pltpu.SemaphoreType.DMA((nsem, NSUB)),
            pltpu.SemaphoreType.DMA((nsem, NSUB)),
            pltpu.SemaphoreType.DMA((nsem, NSUB)),
            pltpu.SemaphoreType.DMA((nsem, NSUB)),
            pltpu.SemaphoreType.DMA((nsem, NSUB)),
            pltpu.SemaphoreType.DMA((NSUB,)),
            pltpu.SemaphoreType.DMA((NSUB,)),
            pltpu.SemaphoreType.REGULAR,
            pltpu.SemaphoreType.REGULAR,
        ],
        compiler_params=pltpu.CompilerParams(
            vmem_limit_bytes=64 * 1024 * 1024,
        ),
    )(sc, x, w_mat)


# device time: 1453550 ns/iter; 1.0333x vs baseline; 1.0021x over previous
import jax
import jax.numpy as jnp
from jax import lax
from jax.experimental import pallas as pl
from jax.experimental.pallas import tpu as pltpu

N_DEV = 16
NSUB = 2


def kernel(x, w_mat, scale_x, scale_w):
    m, _ = x.shape
    _, n = w_mat.shape
    ch = m // N_DEV
    chh = ch // NSUB
    nh = n // 2

    sc = (scale_x * scale_w).astype(jnp.float32).reshape(1, 1)

    def body(sc_ref, x_ref, w_ref, out_ref,
             recv_f, recv_b, send_f, send_b, stg_f, stg_b,
             rss_f, rsr_f, rss_b, rsr_b,
             ags_f, agr_f, ags_b, agr_b,
             stsem_f, stsem_b, cred_f, cred_b):
        me = lax.axis_index("i")
        left = lax.rem(me + N_DEV - 1, N_DEV)
        right = lax.rem(me + 1, N_DEV)

        class Dir:
            pass

        fwd = Dir()
        fwd.dev, fwd.sign, fwd.cred_to, fwd.col = right, -1, left, pl.ds(0, nh)
        fwd.recv, fwd.send, fwd.stg = recv_f, send_f, stg_f
        fwd.rss, fwd.rsr, fwd.ags, fwd.agr = rss_f, rsr_f, ags_f, agr_f
        fwd.stsem, fwd.cred = stsem_f, cred_f
        bwd = Dir()
        bwd.dev, bwd.sign, bwd.cred_to, bwd.col = left, 1, right, pl.ds(nh, nh)
        bwd.recv, bwd.send, bwd.stg = recv_b, send_b, stg_b
        bwd.rss, bwd.rsr, bwd.ags, bwd.agr = rss_b, rsr_b, ags_b, agr_b
        bwd.stsem, bwd.cred = stsem_b, cred_b
        dirs = (fwd, bwd)

        def chunk(k, sign):
            return lax.rem(me + sign * k + 2 * N_DEV, N_DEV)

        def rows(c, sub):
            return pl.ds(c * ch + sub * chh, chh)

        def pgemm(c, sub, D):
            return jax.lax.dot_general(
                x_ref[rows(c, sub), :], w_ref[:, D.col],
                (((1,), (0,)), ((), ())),
                preferred_element_type=jnp.int32)

        def remote(src, dst, ssem, rsem, dev):
            return pltpu.make_async_remote_copy(
                src_ref=src, dst_ref=dst, send_sem=ssem, recv_sem=rsem,
                device_id=(dev,), device_id_type=pl.DeviceIdType.MESH)

        rcur = [[None] * NSUB for _ in dirs]
        for di, D in enumerate(dirs):
            for sub in range(NSUB):
                D.send[sub] = pgemm(chunk(0, D.sign), sub, D)
                r = remote(D.send.at[sub], D.recv.at[0, sub],
                           D.rss.at[0, sub], D.rsr.at[0, sub], D.dev)
                r.start()
                rcur[di][sub] = r

        rag = [[None] * NSUB for _ in dirs]
        sts = [[None] * NSUB for _ in dirs]

        for s in range(N_DEV - 1):
            slot = s % 2
            for sub in range(NSUB):
                for di, D in enumerate(dirs):
                    rcur[di][sub].wait()
                    acc = D.recv[slot, sub] + pgemm(chunk(s + 1, D.sign), sub, D)
                    if s < N_DEV - 2:
                        D.send[sub] = acc
                        if sub == 0 and s + 1 >= 2:
                            pl.semaphore_wait(D.cred, 1)
                        r = remote(D.send.at[sub],
                                   D.recv.at[(s + 1) % 2, sub],
                                   D.rss.at[s + 1, sub],
                                   D.rsr.at[s + 1, sub], D.dev)
                        r.start()
                        rcur[di][sub] = r
                    else:
                        rc = chunk(-1, D.sign)
                        D.stg[sub] = acc.astype(jnp.float32) * sc_ref[0, 0]
                        r = remote(D.stg.at[sub],
                                   out_ref.at[rows(rc, sub), D.col],
                                   D.ags.at[0, sub], D.agr.at[0, sub], D.dev)
                        r.start()
                        rag[di][sub] = r
                        st = pltpu.make_async_copy(
                            D.stg.at[sub], out_ref.at[rows(rc, sub), D.col],
                            D.stsem.at[sub])
                        st.start()
                        sts[di][sub] = st
            for D in dirs:
                if s <= N_DEV - 4:
                    pl.semaphore_signal(D.cred, inc=1,
                                        device_id=(D.cred_to,),
                                        device_id_type=pl.DeviceIdType.MESH)

        for t in range(N_DEV - 1):
            for sub in range(NSUB):
                for di, D in enumerate(dirs):
                    rag[di][sub].wait()
                    if t < N_DEV - 2:
                        cr = rows(chunk(t, D.sign), sub)
                        r = remote(out_ref.at[cr, D.col],
                                   out_ref.at[cr, D.col],
                                   D.ags.at[t + 1, sub],
                                   D.agr.at[t + 1, sub], D.dev)
                        r.start()
                        rag[di][sub] = r
        for di, D in enumerate(dirs):
            for sub in range(NSUB):
                sts[di][sub].wait()

    nsem = N_DEV - 1
    return pl.pallas_call(
        body,
        out_shape=jax.ShapeDtypeStruct((m, n), jnp.float32),
        in_specs=[
            pl.BlockSpec(memory_space=pltpu.SMEM),
            pl.BlockSpec(memory_space=pltpu.VMEM),
            pl.BlockSpec(memory_space=pltpu.VMEM),
        ],
        out_specs=pl.BlockSpec(memory_space=pl.ANY),
        scratch_shapes=[
            pltpu.VMEM((2, NSUB, chh, nh), jnp.int32),
            pltpu.VMEM((2, NSUB, chh, nh), jnp.int32),
            pltpu.VMEM((NSUB, chh, nh), jnp.int32),
            pltpu.VMEM((NSUB, chh, nh), jnp.int32),
            pltpu.VMEM((NSUB, chh, nh), jnp.float32),
            pltpu.VMEM((NSUB, chh, nh), jnp.float32),
            pltpu.SemaphoreType.DMA((nsem, NSUB)),
            pltpu.SemaphoreType.DMA((nsem, NSUB)),
            pltpu.SemaphoreType.DMA((nsem, NSUB)),
            pltpu.SemaphoreType.DMA((nsem, NSUB)),
            pltpu.SemaphoreType.DMA((nsem, NSUB)),
            pltpu.SemaphoreType.DMA((nsem, NSUB)),
            pltpu.SemaphoreType.DMA((nsem, NSUB)),
            pltpu.SemaphoreType.DMA((nsem, NSUB)),
            pltpu.SemaphoreType.DMA((NSUB,)),
            pltpu.SemaphoreType.DMA((NSUB,)),
            pltpu.SemaphoreType.REGULAR,
            pltpu.SemaphoreType.REGULAR,
        ],
        compiler_params=pltpu.CompilerParams(
            vmem_limit_bytes=64 * 1024 * 1024,
        ),
    )(sc, x, w_mat)
